# baseline (speedup 1.0000x reference)
import functools

import jax
import jax.numpy as jnp
from jax import lax
from jax.experimental import pallas as pl
from jax.experimental.pallas import tpu as pltpu
from jax.experimental.pallas import tpu_sc as plsc

D = 64
DP = 128
LANE = 128
G = 4
NBUF = 3
VPAD = 1000064
VB = 128
N_WORKERS = 32


def _transpose_kernel():
    n_blocks = VPAD // VB
    per_w = n_blocks // N_WORKERS
    extra = n_blocks - per_w * N_WORKERS
    mesh = plsc.VectorSubcoreMesh(core_axis_name="c", subcore_axis_name="s")

    @functools.partial(
        pl.kernel,
        mesh=mesh,
        out_type=jax.ShapeDtypeStruct((VPAD, D), jnp.float32),
        scratch_types=[
            pltpu.VMEM((2, D, VB), jnp.float32),
            pltpu.VMEM((2, VB, D), jnp.float32),
            pltpu.SemaphoreType.DMA,
            pltpu.SemaphoreType.DMA,
            pltpu.SemaphoreType.DMA,
            pltpu.SemaphoreType.DMA,
        ],
        compiler_params=pltpu.CompilerParams(use_tc_tiling_on_sc=False,
                                             needs_layout_passes=False),
    )
    def body(tpad_hbm, tlin_hbm, sin, sout, isem0, isem1, osem0, osem1):
        wid = lax.axis_index("s") * 2 + lax.axis_index("c")
        lo = wid * per_w + jnp.minimum(wid, extra)
        n_mine = per_w + jnp.where(wid < extra, 1, 0)
        hi = lo + n_mine
        isems = (isem0, isem1)
        osems = (osem0, osem1)
        iota = lax.iota(jnp.int32, 16)

        def fire_in(b, slot):
            v0 = b * VB
            pltpu.async_copy(tpad_hbm.at[:, pl.ds(v0, VB)], sin.at[slot],
                             isems[slot])

        def wait_in(b, slot):
            v0 = b * VB
            pltpu.make_async_copy(tpad_hbm.at[:, pl.ds(v0, VB)],
                                  sin.at[slot], isems[slot]).wait()

        def fire_out(b, slot):
            v0 = b * VB
            pltpu.async_copy(sout.at[slot], tlin_hbm.at[pl.ds(v0, VB)],
                             osems[slot])

        def wait_out(b, slot):
            v0 = b * VB
            pltpu.make_async_copy(sout.at[slot],
                                  tlin_hbm.at[pl.ds(v0, VB)],
                                  osems[slot]).wait()

        def transpose(slot):
            def step(i, carry):
                for u in range(4):
                    v = i * 4 + u
                    vcol = jnp.full((16,), 0, jnp.int32) + v
                    for di in range(D // 16):
                        vec = plsc.load_gather(
                            sin.at[slot], [di * 16 + iota, vcol])
                        sout[slot, v, pl.ds(di * 16, 16)] = vec
                return carry
            lax.fori_loop(0, VB // 4, step, 0)

        def stage(p, slot):
            @pl.when(p < n_mine)
            def _():
                wait_in(lo + p, slot)
                @pl.when(p >= 2)
                def _():
                    wait_out(lo + p - 2, slot)
                transpose(slot)
                fire_out(lo + p, slot)
                @pl.when(p + 2 < n_mine)
                def _():
                    fire_in(lo + p + 2, slot)

        @pl.when(n_mine > 0)
        def _():
            fire_in(lo, 0)
        @pl.when(n_mine > 1)
        def _():
            fire_in(lo + 1, 1)

        def outer(g, carry):
            stage(2 * g, 0)
            stage(2 * g + 1, 1)
            return carry

        n_outer = (per_w + 1 + 1) // 2 + 1
        lax.fori_loop(0, n_outer, outer, 0)

        @pl.when(n_mine % 2 == 0)
        def _():
            wait_out(hi - 2, 0)
            wait_out(hi - 1, 1)
        @pl.when(n_mine % 2 == 1)
        def _():
            wait_out(hi - 2, 1)
            wait_out(hi - 1, 0)

    return body


def _gather_kernel(n_rows):
    rows_per_w = n_rows // N_WORKERS
    n_chunks = rows_per_w // G
    mesh = plsc.VectorSubcoreMesh(core_axis_name="c", subcore_axis_name="s")

    @functools.partial(
        pl.kernel,
        mesh=mesh,
        out_type=jax.ShapeDtypeStruct((n_rows, LANE, DP), jnp.float32),
        scratch_types=(
            [pltpu.VMEM((rows_per_w, LANE), jnp.int32),
             pltpu.VMEM((NBUF, G, LANE, D), jnp.float32)]
            + [pltpu.SemaphoreType.DMA] * (2 * NBUF)
        ),
        compiler_params=pltpu.CompilerParams(use_tc_tiling_on_sc=False),
    )
    def body(ids_hbm, table_hbm, out_hbm, idx_v, rows_v, *sems):
        gsems = sems[:NBUF]
        ssems = sems[NBUF:]
        wid = lax.axis_index("s") * 2 + lax.axis_index("c")
        base = wid * rows_per_w

        pltpu.sync_copy(ids_hbm.at[pl.ds(base, rows_per_w)], idx_v)

        def fire(k, slot):
            for j in range(G):
                pltpu.async_copy(
                    table_hbm.at[idx_v.at[k * G + j]],
                    rows_v.at[slot, j],
                    gsems[slot],
                )

        def drain_and_store(k, slot):
            for j in range(G):
                pltpu.make_async_copy(
                    table_hbm.at[idx_v.at[k * G + j]],
                    rows_v.at[slot, j],
                    gsems[slot],
                ).wait()
            r0 = base + k * G
            pltpu.async_copy(rows_v.at[slot],
                             out_hbm.at[pl.ds(r0, G), :, pl.ds(0, D)],
                             ssems[slot])

        def wait_store(k, slot):
            r0 = base + k * G
            pltpu.make_async_copy(rows_v.at[slot],
                                  out_hbm.at[pl.ds(r0, G), :, pl.ds(0, D)],
                                  ssems[slot]).wait()

        for k in range(NBUF):
            fire(k, k)
            if k >= 1:
                drain_and_store(k - 1, k - 1)

        def outer(g, carry):
            k0 = g * NBUF
            for b in range(NBUF):
                k = k0 + b
                @pl.when(k - NBUF < n_chunks)
                def _():
                    wait_store(k - NBUF, b)
                @pl.when(k < n_chunks)
                def _():
                    fire(k, b)
                @pl.when(k - 1 < n_chunks)
                def _():
                    drain_and_store(k - 1, (b - 1) % NBUF)
            return carry

        n_groups = -(-(n_chunks + 1 - NBUF) // NBUF)
        lax.fori_loop(1, 1 + n_groups, outer, 0)

        k_last = (1 + n_groups) * NBUF - 1
        for k in range(max(0, k_last - NBUF + 1), n_chunks):
            wait_store(k, k % NBUF)

    return body


def kernel(ids, masks, lengths, extras, table):
    B, L = ids.shape
    n_idx = B * L
    n_rows = n_idx // LANE
    ids2 = ids.reshape(n_rows, LANE)
    tpad = jnp.pad(table.T, ((0, 0), (0, VPAD - table.shape[0])))
    tlin = _transpose_kernel()(tpad)
    out = _gather_kernel(n_rows)(ids2, tlin)
    emb = out.reshape(n_idx, DP)[:, :D].reshape(B, L, D)
    return (emb, masks, lengths, extras)


# speedup vs baseline: 7.3297x; 7.3297x over previous
import functools

import jax
import jax.numpy as jnp
from jax import lax
from jax.experimental import pallas as pl
from jax.experimental.pallas import tpu as pltpu
from jax.experimental.pallas import tpu_sc as plsc

D = 64
DP = 128
LANE = 128
G = 2
NBUF = 5
N_WORKERS = 32


def _gather_kernel(n_rows):
    rows_per_w = n_rows // N_WORKERS
    n_chunks = rows_per_w // G
    mesh = plsc.VectorSubcoreMesh(core_axis_name="c", subcore_axis_name="s")

    @functools.partial(
        pl.kernel,
        mesh=mesh,
        out_type=jax.ShapeDtypeStruct((n_rows, LANE, DP), jnp.float32),
        scratch_types=(
            [pltpu.VMEM((rows_per_w, LANE), jnp.int32),
             pltpu.VMEM((NBUF, G, LANE, D), jnp.float32)]
            + [pltpu.SemaphoreType.DMA] * (2 * NBUF)
        ),
        compiler_params=pltpu.CompilerParams(use_tc_tiling_on_sc=False),
    )
    def body(ids_hbm, table_hbm, out_hbm, idx_v, rows_v, *sems):
        gsems = sems[:NBUF]
        ssems = sems[NBUF:]
        wid = lax.axis_index("s") * 2 + lax.axis_index("c")
        base = wid * rows_per_w

        pltpu.sync_copy(ids_hbm.at[pl.ds(base, rows_per_w)], idx_v)

        def fire(k, slot):
            for j in range(G):
                pltpu.async_copy(
                    table_hbm.at[idx_v.at[k * G + j]],
                    rows_v.at[slot, j],
                    gsems[slot],
                )

        def drain_and_store(k, slot):
            for j in range(G):
                pltpu.make_async_copy(
                    table_hbm.at[idx_v.at[k * G + j]],
                    rows_v.at[slot, j],
                    gsems[slot],
                ).wait()
            r0 = base + k * G
            pltpu.async_copy(rows_v.at[slot],
                             out_hbm.at[pl.ds(r0, G), :, pl.ds(0, D)],
                             ssems[slot])

        def wait_store(k, slot):
            r0 = base + k * G
            pltpu.make_async_copy(rows_v.at[slot],
                                  out_hbm.at[pl.ds(r0, G), :, pl.ds(0, D)],
                                  ssems[slot]).wait()

        for k in range(NBUF):
            fire(k, k)
            if k >= 1:
                drain_and_store(k - 1, k - 1)

        def outer(g, carry):
            k0 = g * NBUF
            for b in range(NBUF):
                k = k0 + b
                @pl.when(k - NBUF < n_chunks)
                def _():
                    wait_store(k - NBUF, b)
                @pl.when(k < n_chunks)
                def _():
                    fire(k, b)
                @pl.when(k - 1 < n_chunks)
                def _():
                    drain_and_store(k - 1, (b - 1) % NBUF)
            return carry

        n_groups = -(-(n_chunks + 1 - NBUF) // NBUF)
        lax.fori_loop(1, 1 + n_groups, outer, 0)

        k_last = (1 + n_groups) * NBUF - 1
        for k in range(max(0, k_last - NBUF + 1), n_chunks):
            wait_store(k, k % NBUF)

    return body


def kernel(ids, masks, lengths, extras, table):
    B, L = ids.shape
    n_idx = B * L
    n_rows = n_idx // LANE
    ids2 = (ids * 2).reshape(n_rows, LANE)
    table_pair = jnp.pad(table, ((0, 0), (0, DP - D))).reshape(-1, D)
    out = _gather_kernel(n_rows)(ids2, table_pair)
    emb = out.reshape(n_idx, DP)[:, :D].reshape(B, L, D)
    return (emb, masks, lengths, extras)
